# baseline (speedup 1.0000x reference)
import functools

import jax
import jax.numpy as jnp
from jax import lax
from jax.experimental import pallas as pl
from jax.experimental.pallas import tpu as pltpu
from jax.experimental.pallas import tpu_sc as plsc

N_IDX = 100000
D = 128
C = 128
NW = 32
N_FULL = N_IDX // C
C_TAIL = N_IDX - N_FULL * C
N_CHUNKS = N_FULL + 1
MAX_CHUNKS_PER_W = -(-N_CHUNKS // NW)

_mesh = plsc.VectorSubcoreMesh(core_axis_name="c", subcore_axis_name="s")


@functools.partial(
    pl.kernel,
    mesh=_mesh,
    out_type=jax.ShapeDtypeStruct((N_IDX, D), jnp.float32),
    scratch_types=[
        pltpu.VMEM((C,), jnp.int32),
        pltpu.VMEM((C, D), jnp.float32),
        pltpu.SemaphoreType.DMA,
    ],
)
def _sc_gather(x_hbm, idx_hbm, out_hbm, idx_v, rows_v, sem):
    w = lax.axis_index("s") * 2 + lax.axis_index("c")

    def body(j, carry):
        g = j * NW + w

        @pl.when(g < N_FULL)
        def _():
            pltpu.sync_copy(idx_hbm.at[pl.ds(g * C, C)], idx_v)
            pltpu.async_copy(x_hbm.at[idx_v], rows_v, sem).wait()
            pltpu.sync_copy(rows_v, out_hbm.at[pl.ds(g * C, C)])

        @pl.when(g == N_FULL)
        def _():
            pltpu.sync_copy(idx_hbm.at[pl.ds(g * C, C_TAIL)],
                            idx_v.at[pl.ds(0, C_TAIL)])
            pltpu.async_copy(x_hbm.at[idx_v.at[pl.ds(0, C_TAIL)]],
                             rows_v.at[pl.ds(0, C_TAIL)], sem).wait()
            pltpu.sync_copy(rows_v.at[pl.ds(0, C_TAIL)],
                            out_hbm.at[pl.ds(g * C, C_TAIL)])

        return carry

    lax.fori_loop(0, MAX_CHUNKS_PER_W, body, 0)


def kernel(x, idx):
    return _sc_gather(x, idx.astype(jnp.int32))


# speedup vs baseline: 1.1792x; 1.1792x over previous
import functools

import jax
import jax.numpy as jnp
from jax import lax
from jax.experimental import pallas as pl
from jax.experimental.pallas import tpu as pltpu
from jax.experimental.pallas import tpu_sc as plsc

N_IDX = 100000
D = 128
C = 128
NW = 32
N_FULL = N_IDX // C
C_TAIL = N_IDX - N_FULL * C
N_SLOTS = 26

_mesh = plsc.VectorSubcoreMesh(core_axis_name="c", subcore_axis_name="s")


@functools.partial(
    pl.kernel,
    mesh=_mesh,
    out_type=jax.ShapeDtypeStruct((N_IDX, D), jnp.float32),
    scratch_types=[
        pltpu.VMEM((C,), jnp.int32),
        pltpu.VMEM((C, D), jnp.float32),
        pltpu.VMEM((C, D), jnp.float32),
        pltpu.SemaphoreType.DMA,
        pltpu.SemaphoreType.DMA,
    ],
)
def _sc_gather(x_hbm, idx_hbm, out_hbm, idx_v, rows_a, rows_b, gsem, osem):
    w = lax.axis_index("s") * 2 + lax.axis_index("c")
    rows = (rows_a, rows_b)

    def drain_full_out():
        pltpu.make_async_copy(rows_a, out_hbm.at[pl.ds(0, C)], osem).wait()

    def slot(j, h):
        g = (2 * h + j) * NW + w
        buf = rows[j % 2]

        @pl.when((g >= 2 * NW) & (g - 2 * NW < N_FULL))
        def _():
            drain_full_out()

        @pl.when(g < N_FULL)
        def _():
            pltpu.sync_copy(idx_hbm.at[pl.ds(g * C, C)], idx_v)
            pltpu.async_copy(x_hbm.at[idx_v], buf, gsem).wait()
            pltpu.async_copy(buf, out_hbm.at[pl.ds(g * C, C)], osem)

        @pl.when(g == N_FULL)
        def _():
            pltpu.sync_copy(idx_hbm.at[pl.ds(g * C, C_TAIL)],
                            idx_v.at[pl.ds(0, C_TAIL)])
            pltpu.async_copy(x_hbm.at[idx_v.at[pl.ds(0, C_TAIL)]],
                             buf.at[pl.ds(0, C_TAIL)], gsem).wait()
            pltpu.async_copy(buf.at[pl.ds(0, C_TAIL)],
                             out_hbm.at[pl.ds(g * C, C_TAIL)], osem)

    def body(h, carry):
        slot(0, h)
        slot(1, h)
        return carry

    lax.fori_loop(0, N_SLOTS // 2, body, 0)

    last_g = (N_SLOTS - 2) * NW + w
    @pl.when(last_g < N_FULL)
    def _():
        drain_full_out()

    @pl.when(last_g == N_FULL)
    def _():
        pltpu.make_async_copy(rows_a.at[pl.ds(0, C_TAIL)],
                              out_hbm.at[pl.ds(0, C_TAIL)], osem).wait()


def kernel(x, idx):
    return _sc_gather(x, idx.astype(jnp.int32))


# speedup vs baseline: 1.3199x; 1.1193x over previous
import functools

import jax
import jax.numpy as jnp
from jax import lax
from jax.experimental import pallas as pl
from jax.experimental.pallas import tpu as pltpu
from jax.experimental.pallas import tpu_sc as plsc

N_IDX = 100000
D = 128
C = 128
NW = 32
N_FULL = N_IDX // C
C_TAIL = N_IDX - N_FULL * C
N_SLOTS = 26
GRID = N_SLOTS * NW * C

_mesh = plsc.VectorSubcoreMesh(core_axis_name="c", subcore_axis_name="s")


@functools.partial(
    pl.kernel,
    mesh=_mesh,
    out_type=jax.ShapeDtypeStruct((N_IDX, D), jnp.float32),
    scratch_types=[
        pltpu.VMEM((N_SLOTS, C), jnp.int32),
        pltpu.VMEM((C, D), jnp.float32),
        pltpu.VMEM((C, D), jnp.float32),
        pltpu.SemaphoreType.DMA,
        pltpu.SemaphoreType.DMA,
    ],
)
def _sc_gather(x_hbm, idx3_hbm, out_hbm, idx_v, rows_a, rows_b, gsem, osem):
    w = lax.axis_index("s") * 2 + lax.axis_index("c")
    rows = (rows_a, rows_b)

    pltpu.sync_copy(idx3_hbm.at[w], idx_v)

    def drain_full_out():
        pltpu.make_async_copy(rows_a, out_hbm.at[pl.ds(0, C)], osem).wait()

    def slot(j, h):
        jj = 2 * h + j
        g = jj * NW + w
        buf = rows[j % 2]

        @pl.when((g >= 2 * NW) & (g - 2 * NW < N_FULL))
        def _():
            drain_full_out()

        @pl.when(g < N_FULL)
        def _():
            pltpu.async_copy(x_hbm.at[idx_v.at[jj]], buf, gsem).wait()
            pltpu.async_copy(buf, out_hbm.at[pl.ds(g * C, C)], osem)

        @pl.when(g == N_FULL)
        def _():
            pltpu.async_copy(x_hbm.at[idx_v.at[jj, pl.ds(0, C_TAIL)]],
                             buf.at[pl.ds(0, C_TAIL)], gsem).wait()
            pltpu.async_copy(buf.at[pl.ds(0, C_TAIL)],
                             out_hbm.at[pl.ds(g * C, C_TAIL)], osem)

    def body(h, carry):
        slot(0, h)
        slot(1, h)
        return carry

    lax.fori_loop(0, N_SLOTS // 2, body, 0)

    last_g = (N_SLOTS - 2) * NW + w
    @pl.when(last_g < N_FULL)
    def _():
        drain_full_out()

    @pl.when(last_g == N_FULL)
    def _():
        pltpu.make_async_copy(rows_a.at[pl.ds(0, C_TAIL)],
                              out_hbm.at[pl.ds(0, C_TAIL)], osem).wait()


def kernel(x, idx):
    idx32 = idx.astype(jnp.int32)
    idx3 = (jnp.zeros((GRID,), jnp.int32).at[:N_IDX].set(idx32)
            .reshape(N_SLOTS, NW, C).transpose(1, 0, 2))
    return _sc_gather(x, idx3)


# speedup vs baseline: 1.3264x; 1.0049x over previous
import functools

import jax
import jax.numpy as jnp
from jax import lax
from jax.experimental import pallas as pl
from jax.experimental.pallas import tpu as pltpu
from jax.experimental.pallas import tpu_sc as plsc

N_IDX = 100000
D = 128
C = 128
NW = 32
N_FULL = N_IDX // C
C_TAIL = N_IDX - N_FULL * C
P_FULL = N_FULL // 2
N_SLOTS = 14
GRID = N_SLOTS * NW * 2 * C

_mesh = plsc.VectorSubcoreMesh(core_axis_name="c", subcore_axis_name="s")


@functools.partial(
    pl.kernel,
    mesh=_mesh,
    out_type=jax.ShapeDtypeStruct((N_IDX, D), jnp.float32),
    scratch_types=[
        pltpu.VMEM((2 * N_SLOTS, C), jnp.int32),
        pltpu.VMEM((2 * C, D), jnp.float32),
        pltpu.VMEM((2 * C, D), jnp.float32),
        pltpu.SemaphoreType.DMA,
        pltpu.SemaphoreType.DMA,
    ],
)
def _sc_gather(x_hbm, idx3_hbm, out_hbm, idx_v, rows_a, rows_b, gsem, osem):
    w = lax.axis_index("s") * 2 + lax.axis_index("c")
    rows = (rows_a, rows_b)

    pltpu.sync_copy(idx3_hbm.at[w], idx_v)

    def drain_out(n_rows):
        pltpu.make_async_copy(rows_a.at[pl.ds(0, n_rows)],
                              out_hbm.at[pl.ds(0, n_rows)], osem).wait()

    def slot(j, h):
        p = 2 * h + j
        q = p * NW + w
        buf = rows[j % 2]

        @pl.when((q >= 2 * NW) & (q - 2 * NW < P_FULL))
        def _():
            drain_out(2 * C)

        @pl.when(q - 2 * NW == P_FULL)
        def _():
            drain_out(C + C_TAIL)

        @pl.when(q < P_FULL)
        def _():
            pltpu.async_copy(x_hbm.at[idx_v.at[2 * p]],
                             buf.at[pl.ds(0, C)], gsem).wait()
            pltpu.async_copy(x_hbm.at[idx_v.at[2 * p + 1]],
                             buf.at[pl.ds(C, C)], gsem).wait()
            pltpu.async_copy(buf, out_hbm.at[pl.ds(q * 2 * C, 2 * C)], osem)

        @pl.when(q == P_FULL)
        def _():
            pltpu.async_copy(x_hbm.at[idx_v.at[2 * p]],
                             buf.at[pl.ds(0, C)], gsem).wait()
            pltpu.async_copy(x_hbm.at[idx_v.at[2 * p + 1, pl.ds(0, C_TAIL)]],
                             buf.at[pl.ds(C, C_TAIL)], gsem).wait()
            pltpu.async_copy(buf.at[pl.ds(0, C + C_TAIL)],
                             out_hbm.at[pl.ds(q * 2 * C, C + C_TAIL)], osem)

    def body(h, carry):
        slot(0, h)
        slot(1, h)
        return carry

    lax.fori_loop(0, N_SLOTS // 2, body, 0)

    for p_last in (N_SLOTS - 2, N_SLOTS - 1):
        q_last = p_last * NW + w

        @pl.when(q_last < P_FULL)
        def _():
            drain_out(2 * C)

        @pl.when(q_last == P_FULL)
        def _():
            drain_out(C + C_TAIL)


def kernel(x, idx):
    idx32 = idx.astype(jnp.int32)
    idx3 = (jnp.zeros((GRID,), jnp.int32).at[:N_IDX].set(idx32)
            .reshape(N_SLOTS, NW, 2 * C).transpose(1, 0, 2)
            .reshape(NW, 2 * N_SLOTS, C))
    return _sc_gather(x, idx3)


# speedup vs baseline: 1.4650x; 1.1045x over previous
import functools

import jax
import jax.numpy as jnp
from jax import lax
from jax.experimental import pallas as pl
from jax.experimental.pallas import tpu as pltpu
from jax.experimental.pallas import tpu_sc as plsc

N_IDX = 100000
D = 128
C = 128
NW = 32
N_FULL = N_IDX // C
C_TAIL = N_IDX - N_FULL * C
P_FULL = N_FULL // 2
N_SLOTS = 14
GRID = N_SLOTS * NW * 2 * C

_mesh = plsc.VectorSubcoreMesh(core_axis_name="c", subcore_axis_name="s")


@functools.partial(
    pl.kernel,
    mesh=_mesh,
    out_type=jax.ShapeDtypeStruct((N_IDX, D), jnp.float32),
    scratch_types=[
        pltpu.VMEM((2 * N_SLOTS, C), jnp.int32),
        pltpu.VMEM((2 * C, D), jnp.float32),
        pltpu.VMEM((2 * C, D), jnp.float32),
        pltpu.SemaphoreType.DMA,
        pltpu.SemaphoreType.DMA,
    ],
)
def _sc_gather(x_hbm, idx3_hbm, out_hbm, idx_v, rows_a, rows_b, gsem, osem):
    w = lax.axis_index("s") * 2 + lax.axis_index("c")
    rows = (rows_a, rows_b)

    pltpu.sync_copy(idx3_hbm.at[w], idx_v)

    def drain_out(n_rows):
        pltpu.make_async_copy(rows_a.at[pl.ds(0, n_rows)],
                              out_hbm.at[pl.ds(0, n_rows)], osem).wait()

    def slot(j, h):
        p = 2 * h + j
        q = p * NW + w
        buf = rows[j % 2]

        @pl.when((q >= 2 * NW) & (q - 2 * NW < P_FULL))
        def _():
            drain_out(2 * C)

        @pl.when(q - 2 * NW == P_FULL)
        def _():
            drain_out(C + C_TAIL)

        @pl.when(q < P_FULL)
        def _():
            g0 = pltpu.async_copy(x_hbm.at[idx_v.at[2 * p]],
                                  buf.at[pl.ds(0, C)], gsem)
            g1 = pltpu.async_copy(x_hbm.at[idx_v.at[2 * p + 1]],
                                  buf.at[pl.ds(C, C)], gsem)
            g0.wait()
            g1.wait()
            pltpu.async_copy(buf, out_hbm.at[pl.ds(q * 2 * C, 2 * C)], osem)

        @pl.when(q == P_FULL)
        def _():
            g0 = pltpu.async_copy(x_hbm.at[idx_v.at[2 * p]],
                                  buf.at[pl.ds(0, C)], gsem)
            g1 = pltpu.async_copy(x_hbm.at[idx_v.at[2 * p + 1, pl.ds(0, C_TAIL)]],
                                  buf.at[pl.ds(C, C_TAIL)], gsem)
            g0.wait()
            g1.wait()
            pltpu.async_copy(buf.at[pl.ds(0, C + C_TAIL)],
                             out_hbm.at[pl.ds(q * 2 * C, C + C_TAIL)], osem)

    def body(h, carry):
        slot(0, h)
        slot(1, h)
        return carry

    lax.fori_loop(0, N_SLOTS // 2, body, 0)

    for p_last in (N_SLOTS - 2, N_SLOTS - 1):
        q_last = p_last * NW + w

        @pl.when(q_last < P_FULL)
        def _():
            drain_out(2 * C)

        @pl.when(q_last == P_FULL)
        def _():
            drain_out(C + C_TAIL)


def kernel(x, idx):
    idx32 = idx.astype(jnp.int32)
    idx3 = (jnp.zeros((GRID,), jnp.int32).at[:N_IDX].set(idx32)
            .reshape(N_SLOTS, NW, 2 * C).transpose(1, 0, 2)
            .reshape(NW, 2 * N_SLOTS, C))
    return _sc_gather(x, idx3)
